# baseline (speedup 1.0000x reference)
import jax
import jax.numpy as jnp
from jax.experimental import pallas as pl

_BETA = 0.25


def _rvq_body(x_ref, cb_ref, embsq_ref, out_ref, loss_ref):
    @pl.when(pl.program_id(0) == 0)
    def _init():
        loss_ref[...] = jnp.zeros_like(loss_ref)

    residual = x_ref[...]
    T = residual.shape[0]
    num_stages, K, _D = cb_ref.shape
    iota_k = jax.lax.broadcasted_iota(jnp.int32, (T, K), 1)
    out = jnp.zeros_like(residual)
    for s in range(num_stages):
        emb = cb_ref[s]
        flatsq = jnp.sum(residual * residual, axis=1, keepdims=True)
        cross = jax.lax.dot_general(
            residual, emb, (((1,), (1,)), ((), ())),
            preferred_element_type=jnp.float32)
        dists = flatsq - 2.0 * cross + embsq_ref[s]
        minv = jnp.min(dists, axis=1, keepdims=True)
        cand = jnp.where(dists == minv, iota_k, K)
        idx = jnp.min(cand, axis=1, keepdims=True)
        onehot = (iota_k == idx).astype(jnp.float32)
        quantized = jax.lax.dot_general(
            onehot, emb, (((1,), (0,)), ((), ())),
            preferred_element_type=jnp.float32,
            precision=jax.lax.Precision.HIGHEST)
        diff = quantized - residual
        loss_ref[s:s + 1, :] += jnp.sum(diff * diff, axis=0, keepdims=True)
        out = out + quantized
        residual = residual - quantized
    out_ref[...] = out


def kernel(x, codebooks):
    B, S, D = x.shape
    num_stages, K, _ = codebooks.shape
    N = B * S
    xf = x.reshape(N, D)
    embsq = jnp.sum(codebooks * codebooks, axis=2)[:, None, :]

    T = 256 if N % 256 == 0 else N
    nblk = N // T
    out_flat, loss_parts = pl.pallas_call(
        _rvq_body,
        grid=(nblk,),
        in_specs=[
            pl.BlockSpec((T, D), lambda i: (i, 0)),
            pl.BlockSpec((num_stages, K, D), lambda i: (0, 0, 0)),
            pl.BlockSpec((num_stages, 1, K), lambda i: (0, 0, 0)),
        ],
        out_specs=[
            pl.BlockSpec((T, D), lambda i: (i, 0)),
            pl.BlockSpec((num_stages, D), lambda i: (0, 0)),
        ],
        out_shape=[
            jax.ShapeDtypeStruct((N, D), jnp.float32),
            jax.ShapeDtypeStruct((num_stages, D), jnp.float32),
        ],
    )(xf, codebooks, embsq)
    out = out_flat.reshape(B, S, D)
    total_loss = (1.0 + _BETA) * jnp.sum(loss_parts) / jnp.float32(N * D)
    return out, total_loss


# speedup vs baseline: 1.4303x; 1.4303x over previous
import jax
import jax.numpy as jnp
from jax.experimental import pallas as pl
from jax.experimental.pallas import tpu as pltpu

_BETA = 0.25


def _rvq_body(x_ref, cb_ref, cbt_ref, embsq_ref, out_ref, loss_ref):
    residual = x_ref[...]
    T = residual.shape[0]
    num_stages, K, D = cb_ref.shape
    iota_k = jax.lax.broadcasted_iota(jnp.int32, (T, K), 1)
    out = jnp.zeros_like(residual)
    loss_rows = []
    for s in range(num_stages):
        flatsq = jnp.sum(residual * residual, axis=1, keepdims=True)
        cross = jax.lax.dot_general(
            residual, cbt_ref[s], (((1,), (0,)), ((), ())),
            preferred_element_type=jnp.float32)
        dists = flatsq - 2.0 * cross + embsq_ref[s]
        minv = jnp.min(dists, axis=1, keepdims=True)
        cand = jnp.where(dists == minv, iota_k, K)
        idx = jnp.min(cand, axis=1, keepdims=True)
        onehot = (iota_k == idx).astype(jnp.float32)
        quantized = jax.lax.dot_general(
            onehot, cb_ref[s], (((1,), (0,)), ((), ())),
            preferred_element_type=jnp.float32,
            precision=jax.lax.Precision.HIGHEST)
        diff = quantized - residual
        loss_rows.append(jnp.sum(diff * diff, axis=0, keepdims=True))
        out = out + quantized
        residual = residual - quantized
    out_ref[...] = out
    loss_ref[0] = jnp.concatenate(loss_rows, axis=0)


def kernel(x, codebooks):
    B, S, D = x.shape
    num_stages, K, _ = codebooks.shape
    N = B * S
    xf = x.reshape(N, D)
    cbt = jnp.transpose(codebooks, (0, 2, 1))
    embsq = jnp.sum(codebooks * codebooks, axis=2)[:, None, :]

    T = 128 if N % 128 == 0 else N
    nblk = N // T
    out_flat, loss_parts = pl.pallas_call(
        _rvq_body,
        grid=(nblk,),
        in_specs=[
            pl.BlockSpec((T, D), lambda i: (i, 0)),
            pl.BlockSpec((num_stages, K, D), lambda i: (0, 0, 0)),
            pl.BlockSpec((num_stages, D, K), lambda i: (0, 0, 0)),
            pl.BlockSpec((num_stages, 1, K), lambda i: (0, 0, 0)),
        ],
        out_specs=[
            pl.BlockSpec((T, D), lambda i: (i, 0)),
            pl.BlockSpec((1, num_stages, D), lambda i: (i, 0, 0)),
        ],
        out_shape=[
            jax.ShapeDtypeStruct((N, D), jnp.float32),
            jax.ShapeDtypeStruct((nblk, num_stages, D), jnp.float32),
        ],
        compiler_params=pltpu.CompilerParams(
            dimension_semantics=("parallel",)),
    )(xf, codebooks, cbt, embsq)
    out = out_flat.reshape(B, S, D)
    total_loss = (1.0 + _BETA) * jnp.sum(loss_parts) / jnp.float32(N * D)
    return out, total_loss


# speedup vs baseline: 6.6736x; 4.6660x over previous
import functools

import jax
import jax.numpy as jnp
from jax import lax
from jax.experimental import pallas as pl
from jax.experimental.pallas import tpu as pltpu
from jax.experimental.pallas import tpu_sc as plsc

_BETA = 0.25


def _stage_body(r_prev_ref, q_prev_ref, cbt_ref, embsq_ref,
                r_ref, idx_ref, loss_ref):
    D = r_prev_ref.shape[1]
    residual = r_prev_ref[...] - q_prev_ref[:, :D]
    r_ref[...] = residual
    loss_ref[0] = jnp.sum(residual * residual, axis=0, keepdims=True)
    T, _D = residual.shape
    K = cbt_ref.shape[1]
    flatsq = jnp.sum(residual * residual, axis=1, keepdims=True)
    cross = lax.dot_general(
        residual, cbt_ref[...], (((1,), (0,)), ((), ())),
        preferred_element_type=jnp.float32)
    dists = flatsq - 2.0 * cross + embsq_ref[0]
    iota_k = lax.broadcasted_iota(jnp.int32, (T, K), 1)
    minv = jnp.min(dists, axis=1, keepdims=True)
    cand = jnp.where(dists == minv, iota_k, K)
    idx_ref[...] = jnp.min(cand, axis=1, keepdims=True)


def _epilogue_body(x_ref, r_ref, q_ref, out_ref, loss_ref):
    r_final = r_ref[...] - q_ref[:, :x_ref.shape[1]]
    loss_ref[0] = jnp.sum(r_final * r_final, axis=0, keepdims=True)
    out_ref[...] = x_ref[...] - r_final


def _make_stage_call(N, D, Dp, K, T):
    nblk = N // T
    return pl.pallas_call(
        _stage_body,
        grid=(nblk,),
        in_specs=[
            pl.BlockSpec((T, D), lambda i: (i, 0)),
            pl.BlockSpec((T, Dp), lambda i: (i, 0)),
            pl.BlockSpec((D, K), lambda i: (0, 0)),
            pl.BlockSpec((1, 1, K), lambda i: (0, 0, 0)),
        ],
        out_specs=[
            pl.BlockSpec((T, D), lambda i: (i, 0)),
            pl.BlockSpec((T, 1), lambda i: (i, 0)),
            pl.BlockSpec((1, 1, D), lambda i: (i, 0, 0)),
        ],
        out_shape=[
            jax.ShapeDtypeStruct((N, D), jnp.float32),
            jax.ShapeDtypeStruct((N, 1), jnp.int32),
            jax.ShapeDtypeStruct((nblk, 1, D), jnp.float32),
        ],
        compiler_params=pltpu.CompilerParams(
            dimension_semantics=("parallel",)),
    )


def _make_epilogue_call(N, D, Dp, T):
    nblk = N // T
    return pl.pallas_call(
        _epilogue_body,
        grid=(nblk,),
        in_specs=[
            pl.BlockSpec((T, D), lambda i: (i, 0)),
            pl.BlockSpec((T, D), lambda i: (i, 0)),
            pl.BlockSpec((T, Dp), lambda i: (i, 0)),
        ],
        out_specs=[
            pl.BlockSpec((T, D), lambda i: (i, 0)),
            pl.BlockSpec((1, 1, D), lambda i: (i, 0, 0)),
        ],
        out_shape=[
            jax.ShapeDtypeStruct((N, D), jnp.float32),
            jax.ShapeDtypeStruct((nblk, 1, D), jnp.float32),
        ],
        compiler_params=pltpu.CompilerParams(
            dimension_semantics=("parallel",)),
    )


def _make_gather(N, Dp, chunk, num_cores, num_subcores):
    num_workers = num_cores * num_subcores
    b_per_w = N // num_workers
    n_chunks = b_per_w // chunk
    n_sub = chunk // 128
    mesh = plsc.VectorSubcoreMesh(core_axis_name="c", subcore_axis_name="s")

    @functools.partial(
        pl.kernel, mesh=mesh,
        out_type=jax.ShapeDtypeStruct((N, Dp), jnp.float32),
        scratch_types=[
            pltpu.VMEM((chunk,), jnp.int32),
            pltpu.VMEM((chunk, Dp), jnp.float32),
            pltpu.SemaphoreType.DMA,
        ],
    )
    def gather_k(cb_hbm, idx_hbm, q_hbm, idx_v, rows_v, sem):
        wid = lax.axis_index("s") * num_cores + lax.axis_index("c")
        base = wid * b_per_w
        for c in range(n_chunks):
            off = base + c * chunk
            pltpu.sync_copy(idx_hbm.at[pl.ds(off, chunk)], idx_v)
            copies = [
                pltpu.async_copy(
                    cb_hbm.at[idx_v.at[pl.ds(j * 128, 128)]],
                    rows_v.at[pl.ds(j * 128, 128)], sem)
                for j in range(n_sub)
            ]
            for cp in copies:
                cp.wait()
            pltpu.sync_copy(rows_v, q_hbm.at[pl.ds(off, chunk)])

    return gather_k


def kernel(x, codebooks):
    B, S, D = x.shape
    num_stages, K, _ = codebooks.shape
    N = B * S
    xf = x.reshape(N, D)
    cbt = jnp.transpose(codebooks, (0, 2, 1))
    embsq = jnp.sum(codebooks * codebooks, axis=2)[:, None, :]

    Dp = 128
    cbp = jnp.pad(codebooks, ((0, 0), (0, 0), (0, Dp - D)))
    info = plsc.get_sparse_core_info()
    gather = _make_gather(N, Dp, 512, info.num_cores, info.num_subcores)
    stage_call = _make_stage_call(N, D, Dp, K, 512)
    epilogue_call = _make_epilogue_call(N, D, Dp, 512)

    r = xf
    q = jnp.zeros((N, Dp), jnp.float32)
    loss_parts = []
    for s in range(num_stages):
        r, idx, lp = stage_call(r, q, cbt[s], embsq[s:s + 1])
        if s > 0:
            loss_parts.append(lp)
        q = gather(cbp[s], idx.reshape(N))
    out_flat, lp_last = epilogue_call(xf, r, q)
    loss_parts.append(lp_last)
    out = out_flat.reshape(B, S, D)
    total_loss = ((1.0 + _BETA) / jnp.float32(N * D)) * jnp.sum(
        jnp.stack([jnp.sum(p) for p in loss_parts]))
    return out, total_loss


# speedup vs baseline: 6.9556x; 1.0423x over previous
import functools

import jax
import jax.numpy as jnp
from jax import lax
from jax.experimental import pallas as pl
from jax.experimental.pallas import tpu as pltpu
from jax.experimental.pallas import tpu_sc as plsc

_BETA = 0.25


def _stage_body(r_prev_ref, q_prev_ref, cbt_ref, embsq_ref,
                r_ref, idx_ref, loss_ref):
    D = r_prev_ref.shape[1]
    residual = r_prev_ref[...] - q_prev_ref[:, :D]
    r_ref[...] = residual
    loss_ref[0] = jnp.sum(residual * residual, axis=0, keepdims=True)
    T, _D = residual.shape
    K = cbt_ref.shape[1]
    flatsq = jnp.sum(residual * residual, axis=1, keepdims=True)
    cross2 = lax.dot_general(
        residual, cbt_ref[...], (((1,), (0,)), ((), ())),
        preferred_element_type=jnp.float32)
    dists = flatsq - cross2 + embsq_ref[0]
    iota_kf = lax.broadcasted_iota(jnp.int32, (T, K), 1).astype(jnp.float32)
    minv = jnp.min(dists, axis=1, keepdims=True)
    cand = jnp.where(dists == minv, iota_kf, jnp.float32(K))
    idx_ref[...] = jnp.min(cand, axis=1, keepdims=True).astype(jnp.int32)


def _epilogue_body(x_ref, r_ref, q_ref, out_ref, loss_ref):
    r_final = r_ref[...] - q_ref[:, :x_ref.shape[1]]
    loss_ref[0] = jnp.sum(r_final * r_final, axis=0, keepdims=True)
    out_ref[...] = x_ref[...] - r_final


def _make_stage_call(N, D, Dp, K, T):
    nblk = N // T
    return pl.pallas_call(
        _stage_body,
        grid=(nblk,),
        in_specs=[
            pl.BlockSpec((T, D), lambda i: (i, 0)),
            pl.BlockSpec((T, Dp), lambda i: (i, 0)),
            pl.BlockSpec((D, K), lambda i: (0, 0)),
            pl.BlockSpec((1, 1, K), lambda i: (0, 0, 0)),
        ],
        out_specs=[
            pl.BlockSpec((T, D), lambda i: (i, 0)),
            pl.BlockSpec((T, 1), lambda i: (i, 0)),
            pl.BlockSpec((1, 1, D), lambda i: (i, 0, 0)),
        ],
        out_shape=[
            jax.ShapeDtypeStruct((N, D), jnp.float32),
            jax.ShapeDtypeStruct((N, 1), jnp.int32),
            jax.ShapeDtypeStruct((nblk, 1, D), jnp.float32),
        ],
        compiler_params=pltpu.CompilerParams(
            dimension_semantics=("parallel",)),
    )


def _make_epilogue_call(N, D, Dp, T):
    nblk = N // T
    return pl.pallas_call(
        _epilogue_body,
        grid=(nblk,),
        in_specs=[
            pl.BlockSpec((T, D), lambda i: (i, 0)),
            pl.BlockSpec((T, D), lambda i: (i, 0)),
            pl.BlockSpec((T, Dp), lambda i: (i, 0)),
        ],
        out_specs=[
            pl.BlockSpec((T, D), lambda i: (i, 0)),
            pl.BlockSpec((1, 1, D), lambda i: (i, 0, 0)),
        ],
        out_shape=[
            jax.ShapeDtypeStruct((N, D), jnp.float32),
            jax.ShapeDtypeStruct((nblk, 1, D), jnp.float32),
        ],
        compiler_params=pltpu.CompilerParams(
            dimension_semantics=("parallel",)),
    )


def _make_gather(N, Dp, chunk, num_cores, num_subcores):
    num_workers = num_cores * num_subcores
    b_per_w = N // num_workers
    n_chunks = b_per_w // chunk
    n_sub = chunk // 128
    mesh = plsc.VectorSubcoreMesh(core_axis_name="c", subcore_axis_name="s")

    @functools.partial(
        pl.kernel, mesh=mesh,
        out_type=jax.ShapeDtypeStruct((N, Dp), jnp.float32),
        scratch_types=[
            pltpu.VMEM((chunk,), jnp.int32),
            pltpu.VMEM((chunk, Dp), jnp.float32),
            pltpu.SemaphoreType.DMA,
        ],
    )
    def gather_k(cb_hbm, idx_hbm, q_hbm, idx_v, rows_v, sem):
        wid = lax.axis_index("s") * num_cores + lax.axis_index("c")
        base = wid * b_per_w
        for c in range(n_chunks):
            off = base + c * chunk
            pltpu.sync_copy(idx_hbm.at[pl.ds(off, chunk)], idx_v)
            copies = [
                pltpu.async_copy(
                    cb_hbm.at[idx_v.at[pl.ds(j * 128, 128)]],
                    rows_v.at[pl.ds(j * 128, 128)], sem)
                for j in range(n_sub)
            ]
            for cp in copies:
                cp.wait()
            pltpu.sync_copy(rows_v, q_hbm.at[pl.ds(off, chunk)])

    return gather_k


def kernel(x, codebooks):
    B, S, D = x.shape
    num_stages, K, _ = codebooks.shape
    N = B * S
    xf = x.reshape(N, D)
    cbt = 2.0 * jnp.transpose(codebooks, (0, 2, 1))
    embsq = jnp.sum(codebooks * codebooks, axis=2)[:, None, :]

    Dp = 128
    cbp = jnp.pad(codebooks, ((0, 0), (0, 0), (0, Dp - D)))
    info = plsc.get_sparse_core_info()
    gather = _make_gather(N, Dp, 512, info.num_cores, info.num_subcores)
    stage_call = _make_stage_call(N, D, Dp, K, 512)
    epilogue_call = _make_epilogue_call(N, D, Dp, 512)

    r = xf
    q = jnp.zeros((N, Dp), jnp.float32)
    loss_parts = []
    for s in range(num_stages):
        r, idx, lp = stage_call(r, q, cbt[s], embsq[s:s + 1])
        if s > 0:
            loss_parts.append(lp)
        q = gather(cbp[s], idx.reshape(N))
    out_flat, lp_last = epilogue_call(xf, r, q)
    loss_parts.append(lp_last)
    out = out_flat.reshape(B, S, D)
    total_loss = ((1.0 + _BETA) / jnp.float32(N * D)) * jnp.sum(
        jnp.stack([jnp.sum(p) for p in loss_parts]))
    return out, total_loss


# speedup vs baseline: 8.7313x; 1.2553x over previous
import functools

import jax
import jax.numpy as jnp
from jax import lax
from jax.experimental import pallas as pl
from jax.experimental.pallas import tpu as pltpu
from jax.experimental.pallas import tpu_sc as plsc

_BETA = 0.25


def _stage_body(r_prev_ref, q_prev_ref, cbt_ref, embsq_ref,
                r_ref, idx_ref, loss_ref):
    D = r_prev_ref.shape[1]
    residual = r_prev_ref[...] - q_prev_ref[:, :D]
    r_ref[...] = residual
    loss_ref[0] = jnp.sum(residual * residual, axis=0, keepdims=True)
    T, _D = residual.shape
    K = cbt_ref.shape[1]
    flatsq = jnp.sum(residual * residual, axis=1, keepdims=True)
    cross2 = lax.dot_general(
        residual, cbt_ref[...], (((1,), (0,)), ((), ())),
        preferred_element_type=jnp.float32)
    embsq = embsq_ref[0]
    n_col = K // 128
    best = (flatsq - cross2[:, 0:128]) + embsq[:, 0:128]
    bcol = jnp.zeros((T, 128), jnp.float32)
    for c in range(1, n_col):
        col = (flatsq - cross2[:, c * 128:(c + 1) * 128]) \
            + embsq[:, c * 128:(c + 1) * 128]
        mask = col < best
        best = jnp.minimum(best, col)
        bcol = jnp.where(mask, jnp.float32(c), bcol)
    minv = jnp.min(best, axis=1, keepdims=True)
    lane_f = lax.broadcasted_iota(jnp.int32, (T, 128), 1).astype(jnp.float32)
    gidx = bcol * 128.0 + lane_f
    cand = jnp.where(best == minv, gidx, jnp.float32(K))
    idx_ref[...] = jnp.min(cand, axis=1, keepdims=True).astype(jnp.int32)


def _epilogue_body(x_ref, r_ref, q_ref, out_ref, loss_ref):
    r_final = r_ref[...] - q_ref[:, :x_ref.shape[1]]
    loss_ref[0] = jnp.sum(r_final * r_final, axis=0, keepdims=True)
    out_ref[...] = x_ref[...] - r_final


def _make_stage_call(N, D, Dp, K, T):
    nblk = N // T
    return pl.pallas_call(
        _stage_body,
        grid=(nblk,),
        in_specs=[
            pl.BlockSpec((T, D), lambda i: (i, 0)),
            pl.BlockSpec((T, Dp), lambda i: (i, 0)),
            pl.BlockSpec((D, K), lambda i: (0, 0)),
            pl.BlockSpec((1, 1, K), lambda i: (0, 0, 0)),
        ],
        out_specs=[
            pl.BlockSpec((T, D), lambda i: (i, 0)),
            pl.BlockSpec((T, 1), lambda i: (i, 0)),
            pl.BlockSpec((1, 1, D), lambda i: (i, 0, 0)),
        ],
        out_shape=[
            jax.ShapeDtypeStruct((N, D), jnp.float32),
            jax.ShapeDtypeStruct((N, 1), jnp.int32),
            jax.ShapeDtypeStruct((nblk, 1, D), jnp.float32),
        ],
        compiler_params=pltpu.CompilerParams(
            dimension_semantics=("parallel",)),
    )


def _make_epilogue_call(N, D, Dp, T):
    nblk = N // T
    return pl.pallas_call(
        _epilogue_body,
        grid=(nblk,),
        in_specs=[
            pl.BlockSpec((T, D), lambda i: (i, 0)),
            pl.BlockSpec((T, D), lambda i: (i, 0)),
            pl.BlockSpec((T, Dp), lambda i: (i, 0)),
        ],
        out_specs=[
            pl.BlockSpec((T, D), lambda i: (i, 0)),
            pl.BlockSpec((1, 1, D), lambda i: (i, 0, 0)),
        ],
        out_shape=[
            jax.ShapeDtypeStruct((N, D), jnp.float32),
            jax.ShapeDtypeStruct((nblk, 1, D), jnp.float32),
        ],
        compiler_params=pltpu.CompilerParams(
            dimension_semantics=("parallel",)),
    )


def _make_gather(N, Dp, chunk, num_cores, num_subcores):
    num_workers = num_cores * num_subcores
    b_per_w = N // num_workers
    n_chunks = b_per_w // chunk
    n_sub = chunk // 128
    mesh = plsc.VectorSubcoreMesh(core_axis_name="c", subcore_axis_name="s")

    @functools.partial(
        pl.kernel, mesh=mesh,
        out_type=jax.ShapeDtypeStruct((N, Dp), jnp.float32),
        scratch_types=[
            pltpu.VMEM((chunk,), jnp.int32),
            pltpu.VMEM((chunk, Dp), jnp.float32),
            pltpu.SemaphoreType.DMA,
        ],
    )
    def gather_k(cb_hbm, idx_hbm, q_hbm, idx_v, rows_v, sem):
        wid = lax.axis_index("s") * num_cores + lax.axis_index("c")
        base = wid * b_per_w
        for c in range(n_chunks):
            off = base + c * chunk
            pltpu.sync_copy(idx_hbm.at[pl.ds(off, chunk)], idx_v)
            copies = [
                pltpu.async_copy(
                    cb_hbm.at[idx_v.at[pl.ds(j * 128, 128)]],
                    rows_v.at[pl.ds(j * 128, 128)], sem)
                for j in range(n_sub)
            ]
            for cp in copies:
                cp.wait()
            pltpu.sync_copy(rows_v, q_hbm.at[pl.ds(off, chunk)])

    return gather_k


def kernel(x, codebooks):
    B, S, D = x.shape
    num_stages, K, _ = codebooks.shape
    N = B * S
    xf = x.reshape(N, D)
    cbt = 2.0 * jnp.transpose(codebooks, (0, 2, 1))
    embsq = jnp.sum(codebooks * codebooks, axis=2)[:, None, :]

    Dp = 128
    cbp = jnp.pad(codebooks, ((0, 0), (0, 0), (0, Dp - D)))
    info = plsc.get_sparse_core_info()
    gather = _make_gather(N, Dp, 512, info.num_cores, info.num_subcores)
    stage_call = _make_stage_call(N, D, Dp, K, 512)
    epilogue_call = _make_epilogue_call(N, D, Dp, 512)

    r = xf
    q = jnp.zeros((N, Dp), jnp.float32)
    loss_parts = []
    for s in range(num_stages):
        r, idx, lp = stage_call(r, q, cbt[s], embsq[s:s + 1])
        if s > 0:
            loss_parts.append(lp)
        q = gather(cbp[s], idx.reshape(N))
    out_flat, lp_last = epilogue_call(xf, r, q)
    loss_parts.append(lp_last)
    out = out_flat.reshape(B, S, D)
    total_loss = ((1.0 + _BETA) / jnp.float32(N * D)) * jnp.sum(
        jnp.stack([jnp.sum(p) for p in loss_parts]))
    return out, total_loss
